# Initial kernel scaffold; baseline (speedup 1.0000x reference)
import jax
import jax.numpy as jnp
from jax.experimental import pallas as pl


def kernel(x, edge_index, batch, W0, b0, gamma0, beta0, W1, b1, gamma1, beta1, W2, b2, gamma2, beta2, cW, cb):
    raise NotImplementedError("write your pallas kernel here")


# speedup vs baseline: 10.6082x; 10.6082x over previous
import functools

import jax
import jax.numpy as jnp
from jax import lax
from jax.experimental import pallas as pl
from jax.experimental.pallas import tpu as pltpu
from jax.experimental.pallas import tpu_sc as plsc

_N = 10000
_E = 320000
_D = 128
_G = 128
_C = 10

_NC = 2
_NS = 16
_NW = _NC * _NS
_EPW = _E // _NW
_CH = 80
_NCHUNK = _EPW // _CH
_GPS = _G // _NS
_ZB = 80
_NZB = _N // _ZB
_PNCH = _N // _CH


@functools.cache
def _sc_kernels():
    mesh = plsc.VectorSubcoreMesh(core_axis_name="c", subcore_axis_name="s")

    deg = functools.partial(
        pl.kernel,
        out_type=jax.ShapeDtypeStruct((_NC, _N, _D), jnp.float32),
        mesh=mesh,
        scratch_types=[
            pltpu.VMEM((_CH,), jnp.int32),
            pltpu.VMEM((_CH, _D), jnp.float32),
            pltpu.VMEM_SHARED((_N, _D), jnp.float32),
        ],
    )(_deg_body)
    prop = functools.partial(
        pl.kernel,
        out_type=jax.ShapeDtypeStruct((_NC, _N, _D), jnp.float32),
        mesh=mesh,
        scratch_types=[
            pltpu.VMEM((_CH,), jnp.int32),
            pltpu.VMEM((_CH,), jnp.int32),
            pltpu.VMEM((_CH, _D), jnp.float32),
            pltpu.VMEM_SHARED((_N, _D), jnp.float32),
            pltpu.SemaphoreType.DMA,
        ],
    )(_prop_body)
    pool = functools.partial(
        pl.kernel,
        out_type=jax.ShapeDtypeStruct((_NC, _G, _D), jnp.float32),
        mesh=mesh,
        scratch_types=[
            pltpu.VMEM((_CH,), jnp.int32),
            pltpu.VMEM((_CH, _D), jnp.float32),
            pltpu.VMEM_SHARED((_G, _D), jnp.float32),
        ],
    )(_pool_body)
    return deg, prop, pool


def _row_blocks(s, fn):
    @pl.loop(0, _NZB // _NS + 1)
    def _(j):
        bid = s + j * _NS

        @pl.when(bid < _NZB)
        def _():
            fn(pl.multiple_of(bid * _ZB, 8))


def _deg_body(dst_hbm, ones_hbm, zeros_hbm, out_hbm, idx_v, ones_v, acc):
    c = lax.axis_index("c")
    s = lax.axis_index("s")
    wid = c * _NS + s
    _row_blocks(s, lambda off: pltpu.sync_copy(
        zeros_hbm.at[pl.ds(off, _ZB)], acc.at[pl.ds(off, _ZB)]))
    pltpu.sync_copy(ones_hbm, ones_v)
    plsc.subcore_barrier()
    base = wid * _EPW

    @pl.loop(0, _NCHUNK)
    def _(i):
        off = pl.multiple_of(base + i * _CH, 8)
        pltpu.sync_copy(dst_hbm.at[pl.ds(off, _CH)], idx_v)
        pltpu.sync_copy(ones_v, acc.at[idx_v], add=True)

    plsc.subcore_barrier()
    _row_blocks(s, lambda off: pltpu.sync_copy(
        acc.at[pl.ds(off, _ZB)], out_hbm.at[c, pl.ds(off, _ZB)]))


def _prop_body(t_hbm, src_hbm, dst_hbm, zeros_hbm, out_hbm,
               sidx, didx, rows, acc, sem):
    c = lax.axis_index("c")
    s = lax.axis_index("s")
    wid = c * _NS + s
    _row_blocks(s, lambda off: pltpu.sync_copy(
        zeros_hbm.at[pl.ds(off, _ZB)], acc.at[pl.ds(off, _ZB)]))
    plsc.subcore_barrier()
    base = wid * _EPW

    @pl.loop(0, _NCHUNK)
    def _(i):
        off = pl.multiple_of(base + i * _CH, 8)
        pltpu.sync_copy(src_hbm.at[pl.ds(off, _CH)], sidx)
        pltpu.sync_copy(dst_hbm.at[pl.ds(off, _CH)], didx)
        pltpu.async_copy(t_hbm.at[sidx], rows, sem).wait()
        pltpu.sync_copy(rows, acc.at[didx], add=True)

    plsc.subcore_barrier()
    _row_blocks(s, lambda off: pltpu.sync_copy(
        acc.at[pl.ds(off, _ZB)], out_hbm.at[c, pl.ds(off, _ZB)]))


def _pool_body(h_hbm, batch_hbm, zeros_hbm, out_hbm, idx_v, rows, acc):
    c = lax.axis_index("c")
    s = lax.axis_index("s")
    wid = c * _NS + s
    pltpu.sync_copy(zeros_hbm.at[pl.ds(s * _GPS, _GPS)],
                    acc.at[pl.ds(s * _GPS, _GPS)])
    plsc.subcore_barrier()
    nloop = _PNCH // _NW + 1

    @pl.loop(0, nloop)
    def _(j):
        cid = wid + j * _NW

        @pl.when(cid < _PNCH)
        def _():
            off = pl.multiple_of(cid * _CH, 8)
            pltpu.sync_copy(batch_hbm.at[pl.ds(off, _CH)], idx_v)
            pltpu.sync_copy(h_hbm.at[pl.ds(off, _CH)], rows)
            pltpu.sync_copy(rows, acc.at[idx_v], add=True)

    plsc.subcore_barrier()
    pltpu.sync_copy(acc.at[pl.ds(s * _GPS, _GPS)],
                    out_hbm.at[c, pl.ds(s * _GPS, _GPS)])


def _dot(a, b):
    return jnp.dot(a, b, preferred_element_type=jnp.float32,
                   precision=lax.Precision.DEFAULT)


def _mm_body(x_ref, w_ref, o_ref):
    o_ref[...] = _dot(x_ref[...], w_ref[...])


_mm_call = pl.pallas_call(
    _mm_body, out_shape=jax.ShapeDtypeStruct((_N, _D), jnp.float32))


def _dinv_scale_body(degacc_ref, xw_ref, dinv_ref, t_ref):
    deg = degacc_ref[0][:, 0:1] + degacc_ref[1][:, 0:1] + 1.0
    dinv = lax.rsqrt(jnp.maximum(deg, 1.0))
    dinv_ref[...] = dinv
    t_ref[...] = xw_ref[...] * dinv


_dinv_scale_call = pl.pallas_call(
    _dinv_scale_body,
    out_shape=(jax.ShapeDtypeStruct((_N, 1), jnp.float32),
               jax.ShapeDtypeStruct((_N, _D), jnp.float32)))


def _postbn(acc_ref, t_ref, dinv_ref, b_ref, g_ref, be_ref):
    u = (acc_ref[0] + acc_ref[1] + t_ref[...]) * dinv_ref[...] + b_ref[...]
    mean = jnp.mean(u, axis=0, keepdims=True)
    var = jnp.mean((u - mean) ** 2, axis=0, keepdims=True)
    return jnp.maximum(
        (u - mean) * lax.rsqrt(var + 1e-5) * g_ref[...] + be_ref[...], 0.0)


def _mid_body(acc_ref, t_ref, dinv_ref, b_ref, g_ref, be_ref, w_ref, o_ref):
    h = _postbn(acc_ref, t_ref, dinv_ref, b_ref, g_ref, be_ref)
    o_ref[...] = _dot(h, w_ref[...]) * dinv_ref[...]


_mid_call = pl.pallas_call(
    _mid_body, out_shape=jax.ShapeDtypeStruct((_N, _D), jnp.float32))


def _last_body(acc_ref, t_ref, dinv_ref, b_ref, g_ref, be_ref, o_ref):
    o_ref[...] = _postbn(acc_ref, t_ref, dinv_ref, b_ref, g_ref, be_ref)


_last_call = pl.pallas_call(
    _last_body, out_shape=jax.ShapeDtypeStruct((_N, _D), jnp.float32))


def _cls_body(p_ref, w_ref, b_ref, o_ref):
    o_ref[...] = _dot(p_ref[0] + p_ref[1], w_ref[...]) + b_ref[...]


_cls_call = pl.pallas_call(
    _cls_body, out_shape=jax.ShapeDtypeStruct((_G, _C), jnp.float32))


def kernel(x, edge_index, batch, W0, b0, gamma0, beta0, W1, b1, gamma1,
           beta1, W2, b2, gamma2, beta2, cW, cb):
    src = edge_index[0]
    dst = edge_index[1]
    zeros_nd = jnp.zeros((_N, _D), jnp.float32)
    ones_ch = jnp.ones((_CH, _D), jnp.float32)

    deg_kernel, prop_kernel, pool_kernel = _sc_kernels()

    degacc = deg_kernel(dst, ones_ch, zeros_nd)
    xw0 = _mm_call(x, W0)
    dinv, t = _dinv_scale_call(degacc, xw0)

    for (b, g, be, Wn) in ((b0, gamma0, beta0, W1), (b1, gamma1, beta1, W2)):
        acc = prop_kernel(t, src, dst, zeros_nd)
        t = _mid_call(acc, t, dinv, b.reshape(1, _D), g.reshape(1, _D),
                      be.reshape(1, _D), Wn)
    acc = prop_kernel(t, src, dst, zeros_nd)
    h = _last_call(acc, t, dinv, b2.reshape(1, _D), gamma2.reshape(1, _D),
                   beta2.reshape(1, _D))
    pacc = pool_kernel(h, batch, zeros_nd)
    return _cls_call(pacc, cW, cb.reshape(1, _C))
